# baseline (speedup 1.0000x reference)
import functools
import math

import jax
import jax.numpy as jnp
from jax import lax
from jax.experimental import pallas as pl
from jax.experimental.pallas import tpu as pltpu
from jax.experimental.pallas import tpu_sc as plsc

_NC = 2
_NS = 16
_NW = _NC * _NS
_L = 16


def _pe_tc_body(out_ref):
    s, d = out_ref.shape
    j = lax.broadcasted_iota(jnp.int32, (s, d), 1)
    pos = lax.broadcasted_iota(jnp.int32, (s, d), 0).astype(jnp.float32) + 1.0
    jeven = (j - (j % 2)).astype(jnp.float32)
    inv_divisor = jnp.exp(jeven * (-math.log(10000.0) / d))
    angle = pos * inv_divisor
    out_ref[...] = jnp.where(j % 2 == 0, jnp.sin(angle), jnp.cos(angle))


def _position_encoding(seq, hidden):
    return pl.pallas_call(
        _pe_tc_body,
        out_shape=jax.ShapeDtypeStruct((seq, hidden), jnp.float32),
    )()


def _sc_body(seq, hidden, bpw, idx_hbm, table_hbm, pe_hbm, out_hbm,
             idx_v, pe_v, gbufs, obufs, gsems, osems):
    cid = lax.axis_index("c")
    sid = lax.axis_index("s")
    wid = sid * _NC + cid
    hb = hidden // 8

    pltpu.sync_copy(pe_hbm, pe_v)
    pltpu.sync_copy(idx_hbm.at[:, pl.ds(wid * bpw, bpw)], idx_v)

    iota = lax.broadcasted_iota(jnp.int32, (_L,), 0)
    row_idx = [iota + bb * _L for bb in range(bpw // _L)]

    def gather(s, b):
        pltpu.async_copy(table_hbm.at[idx_v.at[s]], gbufs[b], gsems[b])

    def wait_gather(s, b):
        pltpu.make_async_copy(table_hbm.at[idx_v.at[s]], gbufs[b],
                              gsems[b]).wait()

    def put(s, b):
        pltpu.async_copy(obufs[b], out_hbm.at[s, :, wid], osems[b])

    def wait_put(s, b):
        pltpu.make_async_copy(obufs[b], out_hbm.at[s, :, wid],
                              osems[b]).wait()

    def transpose_pe(s, b):
        gbuf, obuf = gbufs[b], obufs[b]

        def hbody(h, carry):
            pe_b = plsc.load_gather(
                pe_v, [jnp.full((_L,), s, jnp.int32),
                       jnp.full((_L,), h, jnp.int32)])
            h8 = h // 8
            hr = h % 8
            hsplat = jnp.full((_L,), h, jnp.int32)
            for bb in range(len(row_idx)):
                t = plsc.load_gather(gbuf, [row_idx[bb], hsplat])
                obuf[h8, hr, pl.ds(bb * _L, _L)] = t + pe_b
            return carry

        lax.fori_loop(0, hidden, hbody, 0, unroll=4)

    gather(0, 0)

    def group(g, carry):
        for b in range(2):
            s = g * 2 + b

            @pl.when(s + 1 < seq)
            def _():
                gather(s + 1, 1 - b)

            wait_gather(s, b)

            @pl.when(s >= 2)
            def _():
                wait_put(s - 2, b)

            transpose_pe(s, b)
            put(s, b)
        return carry

    lax.fori_loop(0, seq // 2, group, 0, unroll=False)

    for s in (seq - 2, seq - 1):
        wait_put(s, s % 2)


def _sc_gather(idx_t, table, pe):
    seq, batch = idx_t.shape
    hidden = table.shape[1]
    bpw = batch // _NW
    mesh = plsc.VectorSubcoreMesh(
        core_axis_name="c", subcore_axis_name="s",
        num_cores=_NC, num_subcores=_NS)
    body = functools.partial(_sc_body, seq, hidden, bpw)
    return pl.kernel(
        body,
        out_type=jax.ShapeDtypeStruct((seq, hidden // 8, _NW, 8, 128),
                                      jnp.float32),
        mesh=mesh,
        scratch_types=[
            pltpu.VMEM((seq, bpw), jnp.int32),
            pltpu.VMEM((seq, hidden), jnp.float32),
            [pltpu.VMEM((bpw, hidden), jnp.float32) for _ in range(2)],
            [pltpu.VMEM((hidden // 8, 8, 128), jnp.float32)
             for _ in range(2)],
            [pltpu.SemaphoreType.DMA for _ in range(2)],
            [pltpu.SemaphoreType.DMA for _ in range(2)],
        ],
        compiler_params=pltpu.CompilerParams(use_tc_tiling_on_sc=False,
                                             needs_layout_passes=False),
    )(idx_t, table, pe)


def kernel(inputs, table):
    batch, seq = inputs.shape
    hidden = table.shape[1]
    pe = _position_encoding(seq, hidden)
    idx_t = inputs.T
    out5 = _sc_gather(idx_t, table, pe)
    out = out5.transpose(2, 4, 0, 1, 3).reshape(batch, seq, hidden)
    return out


# speedup vs baseline: 1.6110x; 1.6110x over previous
import functools
import math

import jax
import jax.numpy as jnp
from jax import lax
from jax.experimental import pallas as pl
from jax.experimental.pallas import tpu as pltpu
from jax.experimental.pallas import tpu_sc as plsc

_NC = 2
_NS = 16
_NW = _NC * _NS
_L = 16


def _pe_tc_body(out_ref):
    s, d = out_ref.shape
    j = lax.broadcasted_iota(jnp.int32, (s, d), 1)
    pos = lax.broadcasted_iota(jnp.int32, (s, d), 0).astype(jnp.float32) + 1.0
    jeven = (j - (j % 2)).astype(jnp.float32)
    inv_divisor = jnp.exp(jeven * (-math.log(10000.0) / d))
    angle = pos * inv_divisor
    out_ref[...] = jnp.where(j % 2 == 0, jnp.sin(angle), jnp.cos(angle))


def _position_encoding(seq, hidden):
    return pl.pallas_call(
        _pe_tc_body,
        out_shape=jax.ShapeDtypeStruct((seq, hidden), jnp.float32),
    )()


def _sc_body(seq, hidden, bpw, idx_hbm, table_hbm, pe_hbm, out_hbm,
             idx_v, pe_v, gbufs, obufs, gsems, osems):
    cid = lax.axis_index("c")
    sid = lax.axis_index("s")
    wid = sid * _NC + cid
    hb = hidden // 8

    pltpu.sync_copy(pe_hbm, pe_v)
    pltpu.sync_copy(idx_hbm.at[:, pl.ds(wid * bpw, bpw)], idx_v)

    iota = lax.broadcasted_iota(jnp.int32, (_L,), 0)
    row_idx = [iota + bb * _L for bb in range(bpw // _L)]

    def gather(s, b):
        pltpu.async_copy(table_hbm.at[idx_v.at[s]], gbufs[b], gsems[b])

    def wait_gather(s, b):
        pltpu.make_async_copy(table_hbm.at[idx_v.at[s]], gbufs[b],
                              gsems[b]).wait()

    def put(s, b):
        pltpu.async_copy(obufs[b], out_hbm.at[s, :, wid], osems[b])

    def wait_put(s, b):
        pltpu.make_async_copy(obufs[b], out_hbm.at[s, :, wid],
                              osems[b]).wait()

    def transpose_pe(s, b):
        gbuf, obuf = gbufs[b], obufs[b]
        pe_base = s * hidden

        @plsc.parallel_loop(0, hb)
        def _(h8):
            for hr in range(8):
                h = h8 * 8 + hr
                csplat = jnp.full((_L,), h, jnp.int32)
                pe_b = plsc.load_gather(
                    pe_v, [jnp.full((_L,), pe_base + h, jnp.int32)])
                for bb in range(len(row_idx)):
                    t = plsc.load_gather(gbuf, [row_idx[bb], csplat])
                    obuf[h8, hr, pl.ds(bb * _L, _L)] = t + pe_b

    gather(0, 0)

    def group(g, carry):
        for b in range(2):
            s = g * 2 + b

            @pl.when(s + 1 < seq)
            def _():
                gather(s + 1, 1 - b)

            wait_gather(s, b)

            @pl.when(s >= 2)
            def _():
                wait_put(s - 2, b)

            transpose_pe(s, b)
            put(s, b)
        return carry

    lax.fori_loop(0, seq // 2, group, 0, unroll=False)

    for s in (seq - 2, seq - 1):
        wait_put(s, s % 2)


def _sc_gather(idx_t, table, pe):
    seq, batch = idx_t.shape
    hidden = table.shape[1]
    bpw = batch // _NW
    mesh = plsc.VectorSubcoreMesh(
        core_axis_name="c", subcore_axis_name="s",
        num_cores=_NC, num_subcores=_NS)
    body = functools.partial(_sc_body, seq, hidden, bpw)
    return pl.kernel(
        body,
        out_type=jax.ShapeDtypeStruct((seq, hidden // 8, _NW, 8, 128),
                                      jnp.float32),
        mesh=mesh,
        scratch_types=[
            pltpu.VMEM((seq, bpw), jnp.int32),
            pltpu.VMEM((seq * hidden,), jnp.float32),
            [pltpu.VMEM((bpw, hidden), jnp.float32) for _ in range(2)],
            [pltpu.VMEM((hidden // 8, 8, 128), jnp.float32)
             for _ in range(2)],
            [pltpu.SemaphoreType.DMA for _ in range(2)],
            [pltpu.SemaphoreType.DMA for _ in range(2)],
        ],
        compiler_params=pltpu.CompilerParams(use_tc_tiling_on_sc=False,
                                             needs_layout_passes=False),
    )(idx_t, table, pe)


def kernel(inputs, table):
    batch, seq = inputs.shape
    hidden = table.shape[1]
    pe = _position_encoding(seq, hidden)
    idx_t = inputs.T
    out5 = _sc_gather(idx_t, table, pe.reshape(-1))
    out = out5.transpose(2, 4, 0, 1, 3).reshape(batch, seq, hidden)
    return out


# speedup vs baseline: 5.1637x; 3.2053x over previous
import functools
import math

import jax
import jax.numpy as jnp
from jax import lax
from jax.experimental import pallas as pl
from jax.experimental.pallas import tpu as pltpu
from jax.experimental.pallas import tpu_sc as plsc

_NC = 2
_NS = 16
_NW = _NC * _NS
_L = 16


def _pe_tc_body(hidden, out_ref):
    rows, cols = out_ref.shape
    r = lax.broadcasted_iota(jnp.int32, (rows, cols), 0)
    c = lax.broadcasted_iota(jnp.int32, (rows, cols), 1)
    i = r * cols + c
    h = i % hidden
    pos = (i // hidden + 1).astype(jnp.float32)
    jeven = (h - (h % 2)).astype(jnp.float32)
    inv_divisor = jnp.exp(jeven * (-math.log(10000.0) / hidden))
    angle = pos * inv_divisor
    out_ref[...] = jnp.where(h % 2 == 0, jnp.sin(angle), jnp.cos(angle))


def _position_encoding(seq, hidden):
    rows = seq * hidden // 128
    return pl.pallas_call(
        functools.partial(_pe_tc_body, hidden),
        out_shape=jax.ShapeDtypeStruct((rows, 128), jnp.float32),
    )()


def _sc_body(seq, hidden, bpw, idx_hbm, table_hbm, pe_hbm, out_hbm,
             idx_v, pbufs, gbufs, obufs, gsems, osems):
    cid = lax.axis_index("c")
    sid = lax.axis_index("s")
    wid = sid * _NC + cid
    hb = hidden // 8
    nh = hidden // _L

    pltpu.sync_copy(idx_hbm.at[:, pl.ds(wid * bpw, bpw)], idx_v)

    iota = lax.broadcasted_iota(jnp.int32, (_L,), 0)
    hvecs = [hh * _L + iota for hh in range(nh)]

    def pe_slice(s):
        return pe_hbm.at[s * hidden // 128, pl.ds(s * hidden % 128, hidden)]

    def gather(s, b):
        pltpu.async_copy(table_hbm.at[idx_v.at[s]], gbufs[b], gsems[b])
        pltpu.async_copy(pe_slice(s), pbufs[b], gsems[b])

    def wait_gather(s, b):
        pltpu.make_async_copy(table_hbm.at[idx_v.at[s]], gbufs[b],
                              gsems[b]).wait()
        pltpu.make_async_copy(pe_slice(s), pbufs[b], gsems[b]).wait()

    def put(s, b):
        for h8 in range(hb):
            pltpu.async_copy(obufs[b].at[pl.ds(h8 * 8, 8), pl.ds(0, 128)],
                             out_hbm.at[s, h8, wid], osems[b])

    def wait_put(s, b):
        for h8 in range(hb):
            pltpu.make_async_copy(
                obufs[b].at[pl.ds(h8 * 8, 8), pl.ds(0, 128)],
                out_hbm.at[s, h8, wid], osems[b]).wait()

    def transpose_pe(s, b):
        gbuf, obuf, pbuf = gbufs[b], obufs[b], pbufs[b]
        pes = [pbuf[pl.ds(hh * _L, _L)] for hh in range(nh)]

        @plsc.parallel_loop(0, bpw)
        def _(r):
            cvec = jnp.full((_L,), r, jnp.int32)
            for hh in range(nh):
                v = gbuf[r, pl.ds(hh * _L, _L)]
                plsc.store_scatter(obuf, [hvecs[hh], cvec], v + pes[hh])

    gather(0, 0)

    def group(g, carry):
        for b in range(2):
            s = g * 2 + b

            @pl.when(s + 1 < seq)
            def _():
                gather(s + 1, 1 - b)

            wait_gather(s, b)

            @pl.when(s >= 2)
            def _():
                wait_put(s - 2, b)

            transpose_pe(s, b)
            put(s, b)
        return carry

    lax.fori_loop(0, seq // 2, group, 0, unroll=False)

    for s in (seq - 2, seq - 1):
        wait_put(s, s % 2)


def _sc_gather(idx_t, table, pe):
    seq, batch = idx_t.shape
    hidden = table.shape[1]
    bpw = batch // _NW
    mesh = plsc.VectorSubcoreMesh(
        core_axis_name="c", subcore_axis_name="s",
        num_cores=_NC, num_subcores=_NS)
    body = functools.partial(_sc_body, seq, hidden, bpw)
    return pl.kernel(
        body,
        out_type=jax.ShapeDtypeStruct((seq, hidden // 8, _NW, 8, 128),
                                      jnp.float32),
        mesh=mesh,
        scratch_types=[
            pltpu.VMEM((seq, bpw), jnp.int32),
            [pltpu.VMEM((hidden,), jnp.float32) for _ in range(2)],
            [pltpu.VMEM((bpw, hidden), jnp.float32) for _ in range(2)],
            [pltpu.VMEM((hidden, 129), jnp.float32) for _ in range(2)],
            [pltpu.SemaphoreType.DMA for _ in range(2)],
            [pltpu.SemaphoreType.DMA for _ in range(2)],
        ],
        compiler_params=pltpu.CompilerParams(use_tc_tiling_on_sc=False,
                                             needs_layout_passes=False),
    )(idx_t, table, pe)


def kernel(inputs, table):
    batch, seq = inputs.shape
    hidden = table.shape[1]
    pe = _position_encoding(seq, hidden)
    idx_t = inputs.T
    out5 = _sc_gather(idx_t, table, pe)
    out = out5.transpose(2, 4, 0, 1, 3).reshape(batch, seq, hidden)
    return out


# speedup vs baseline: 5.9586x; 1.1540x over previous
import functools
import math

import jax
import jax.numpy as jnp
from jax import lax
from jax.experimental import pallas as pl
from jax.experimental.pallas import tpu as pltpu
from jax.experimental.pallas import tpu_sc as plsc

_NC = 2
_NS = 16
_NW = _NC * _NS
_L = 16


def _pe_tc_body(hidden, out_ref):
    rows, cols = out_ref.shape
    r = lax.broadcasted_iota(jnp.int32, (rows, cols), 0)
    c = lax.broadcasted_iota(jnp.int32, (rows, cols), 1)
    i = r * cols + c
    h = i % hidden
    pos = (i // hidden + 1).astype(jnp.float32)
    jeven = (h - (h % 2)).astype(jnp.float32)
    inv_divisor = jnp.exp(jeven * (-math.log(10000.0) / hidden))
    angle = pos * inv_divisor
    out_ref[...] = jnp.where(h % 2 == 0, jnp.sin(angle), jnp.cos(angle))


def _position_encoding(seq, hidden):
    rows = seq * hidden // 128
    return pl.pallas_call(
        functools.partial(_pe_tc_body, hidden),
        out_shape=jax.ShapeDtypeStruct((rows, 128), jnp.float32),
    )()


def _sc_body(seq, hidden, bpw, idx_hbm, table_hbm, pe_hbm, out_hbm,
             idx_v, pbufs, gbufs, obufs, gsems, osems):
    cid = lax.axis_index("c")
    sid = lax.axis_index("s")
    wid = sid * _NC + cid
    hb = hidden // 8
    nh = hidden // _L

    pltpu.sync_copy(idx_hbm.at[:, pl.ds(wid * bpw, bpw)], idx_v)

    iota = lax.broadcasted_iota(jnp.int32, (_L,), 0)
    hvecs = [hh * _L + iota for hh in range(nh)]

    def pe_slice(s):
        return pe_hbm.at[s * hidden // 128, pl.ds(s * hidden % 128, hidden)]

    def gather(s, b):
        pltpu.async_copy(table_hbm.at[idx_v.at[s]], gbufs[b], gsems[b])
        pltpu.async_copy(pe_slice(s), pbufs[b], gsems[b])

    def wait_gather(s, b):
        pltpu.make_async_copy(table_hbm.at[idx_v.at[s]], gbufs[b],
                              gsems[b]).wait()
        pltpu.make_async_copy(pe_slice(s), pbufs[b], gsems[b]).wait()

    def put(s, b):
        for h8 in range(hb):
            pltpu.async_copy(obufs[b].at[pl.ds(h8 * 8, 8), pl.ds(0, 128)],
                             out_hbm.at[s, h8, wid], osems[b])

    def wait_put(s, b):
        for h8 in range(hb):
            pltpu.make_async_copy(
                obufs[b].at[pl.ds(h8 * 8, 8), pl.ds(0, 128)],
                out_hbm.at[s, h8, wid], osems[b]).wait()

    def transpose_pe(s, b):
        gbuf, obuf, pbuf = gbufs[b], obufs[b], pbufs[b]
        pes = [pbuf[pl.ds(hh * _L, _L)] for hh in range(nh)]

        @plsc.parallel_loop(0, bpw)
        def _(r):
            cvec = jnp.full((_L,), r, jnp.int32)
            for hh in range(nh):
                v = gbuf[r, pl.ds(hh * _L, _L)]
                plsc.store_scatter(obuf, [hvecs[hh], cvec], v + pes[hh])

    nbuf, lead = 4, 2
    for s0 in range(lead):
        gather(s0, s0)

    def group(g, carry):
        for b in range(nbuf):
            s = g * nbuf + b

            @pl.when(s + lead < seq)
            def _():
                gather(s + lead, (b + lead) % nbuf)

            wait_gather(s, b)

            @pl.when(s >= nbuf)
            def _():
                wait_put(s - nbuf, b)

            transpose_pe(s, b)
            put(s, b)
        return carry

    lax.fori_loop(0, seq // nbuf, group, 0, unroll=False)

    for s in range(seq - nbuf, seq):
        wait_put(s, s % nbuf)


def _sc_gather(idx_t, table, pe):
    seq, batch = idx_t.shape
    hidden = table.shape[1]
    bpw = batch // _NW
    mesh = plsc.VectorSubcoreMesh(
        core_axis_name="c", subcore_axis_name="s",
        num_cores=_NC, num_subcores=_NS)
    body = functools.partial(_sc_body, seq, hidden, bpw)
    return pl.kernel(
        body,
        out_type=jax.ShapeDtypeStruct((seq, hidden // 8, _NW, 8, 128),
                                      jnp.float32),
        mesh=mesh,
        scratch_types=[
            pltpu.VMEM((seq, bpw), jnp.int32),
            [pltpu.VMEM((hidden,), jnp.float32) for _ in range(4)],
            [pltpu.VMEM((bpw, hidden), jnp.float32) for _ in range(4)],
            [pltpu.VMEM((hidden, 129), jnp.float32) for _ in range(4)],
            [pltpu.SemaphoreType.DMA for _ in range(4)],
            [pltpu.SemaphoreType.DMA for _ in range(4)],
        ],
        compiler_params=pltpu.CompilerParams(use_tc_tiling_on_sc=False,
                                             needs_layout_passes=False),
    )(idx_t, table, pe)


def kernel(inputs, table):
    batch, seq = inputs.shape
    hidden = table.shape[1]
    pe = _position_encoding(seq, hidden)
    idx_t = inputs.T
    out5 = _sc_gather(idx_t, table, pe)
    out = out5.transpose(2, 4, 0, 1, 3).reshape(batch, seq, hidden)
    return out


# speedup vs baseline: 6.0073x; 1.0082x over previous
import functools
import math

import jax
import jax.numpy as jnp
from jax import lax
from jax.experimental import pallas as pl
from jax.experimental.pallas import tpu as pltpu
from jax.experimental.pallas import tpu_sc as plsc

_NC = 2
_NS = 16
_NW = _NC * _NS
_L = 16


def _pe_tc_body(hidden, out_ref):
    rows, cols = out_ref.shape
    r = lax.broadcasted_iota(jnp.int32, (rows, cols), 0)
    c = lax.broadcasted_iota(jnp.int32, (rows, cols), 1)
    i = r * cols + c
    h = i % hidden
    pos = (i // hidden + 1).astype(jnp.float32)
    jeven = (h - (h % 2)).astype(jnp.float32)
    inv_divisor = jnp.exp(jeven * (-math.log(10000.0) / hidden))
    angle = pos * inv_divisor
    out_ref[...] = jnp.where(h % 2 == 0, jnp.sin(angle), jnp.cos(angle))


def _position_encoding(seq, hidden):
    rows = seq * hidden // 128
    return pl.pallas_call(
        functools.partial(_pe_tc_body, hidden),
        out_shape=jax.ShapeDtypeStruct((rows, 128), jnp.float32),
    )()


def _sc_body(seq, hidden, bpw, idx_hbm, table_hbm, pe_hbm, out_hbm,
             idx_v, pbufs, gbufs, obufs, gsems, osems):
    cid = lax.axis_index("c")
    sid = lax.axis_index("s")
    wid = sid * _NC + cid
    hb = hidden // 8
    nh = hidden // _L

    pltpu.sync_copy(idx_hbm.at[:, pl.ds(wid * bpw, bpw)], idx_v)

    iota = lax.broadcasted_iota(jnp.int32, (_L,), 0)
    hvecs = [hh * _L + iota for hh in range(nh)]
    h8vecs = [hv // 8 for hv in hvecs]
    hrvecs = [hv % 8 for hv in hvecs]

    def pe_slice(s):
        return pe_hbm.at[s * hidden // 128, pl.ds(s * hidden % 128, hidden)]

    def gather(s, b):
        pltpu.async_copy(table_hbm.at[idx_v.at[s]], gbufs[b], gsems[b])
        pltpu.async_copy(pe_slice(s), pbufs[b], gsems[b])

    def wait_gather(s, b):
        pltpu.make_async_copy(table_hbm.at[idx_v.at[s]], gbufs[b],
                              gsems[b]).wait()
        pltpu.make_async_copy(pe_slice(s), pbufs[b], gsems[b]).wait()

    def put(s, b):
        pltpu.async_copy(obufs[b].at[:, :, pl.ds(0, 128)],
                         out_hbm.at[s, :, wid], osems[b])

    def wait_put(s, b):
        pltpu.make_async_copy(obufs[b].at[:, :, pl.ds(0, 128)],
                              out_hbm.at[s, :, wid], osems[b]).wait()

    def transpose_pe(s, b):
        gbuf, obuf, pbuf = gbufs[b], obufs[b], pbufs[b]
        pes = [pbuf[pl.ds(hh * _L, _L)] for hh in range(nh)]

        @plsc.parallel_loop(0, bpw)
        def _(r):
            cvec = jnp.full((_L,), r, jnp.int32)
            for hh in range(nh):
                v = gbuf[r, pl.ds(hh * _L, _L)]
                plsc.store_scatter(obuf, [h8vecs[hh], hrvecs[hh], cvec],
                                   v + pes[hh])

    nbuf, lead = 4, 2
    for s0 in range(lead):
        gather(s0, s0)

    def group(g, carry):
        for b in range(nbuf):
            s = g * nbuf + b

            @pl.when(s + lead < seq)
            def _():
                gather(s + lead, (b + lead) % nbuf)

            wait_gather(s, b)

            @pl.when(s >= nbuf)
            def _():
                wait_put(s - nbuf, b)

            transpose_pe(s, b)
            put(s, b)
        return carry

    lax.fori_loop(0, seq // nbuf, group, 0, unroll=False)

    for s in range(seq - nbuf, seq):
        wait_put(s, s % nbuf)


def _sc_gather(idx_t, table, pe):
    seq, batch = idx_t.shape
    hidden = table.shape[1]
    bpw = batch // _NW
    mesh = plsc.VectorSubcoreMesh(
        core_axis_name="c", subcore_axis_name="s",
        num_cores=_NC, num_subcores=_NS)
    body = functools.partial(_sc_body, seq, hidden, bpw)
    return pl.kernel(
        body,
        out_type=jax.ShapeDtypeStruct((seq, hidden // 8, _NW, 8, 128),
                                      jnp.float32),
        mesh=mesh,
        scratch_types=[
            pltpu.VMEM((seq, bpw), jnp.int32),
            [pltpu.VMEM((hidden,), jnp.float32) for _ in range(4)],
            [pltpu.VMEM((bpw, hidden), jnp.float32) for _ in range(4)],
            [pltpu.VMEM((hidden // 8, 8, 129), jnp.float32)
             for _ in range(4)],
            [pltpu.SemaphoreType.DMA for _ in range(4)],
            [pltpu.SemaphoreType.DMA for _ in range(4)],
        ],
        compiler_params=pltpu.CompilerParams(use_tc_tiling_on_sc=False,
                                             needs_layout_passes=False),
    )(idx_t, table, pe)


def kernel(inputs, table):
    batch, seq = inputs.shape
    hidden = table.shape[1]
    pe = _position_encoding(seq, hidden)
    idx_t = inputs.T
    out5 = _sc_gather(idx_t, table, pe)
    out = out5.transpose(2, 4, 0, 1, 3).reshape(batch, seq, hidden)
    return out


# speedup vs baseline: 6.2086x; 1.0335x over previous
import functools
import math

import jax
import jax.numpy as jnp
from jax import lax
from jax.experimental import pallas as pl
from jax.experimental.pallas import tpu as pltpu
from jax.experimental.pallas import tpu_sc as plsc

_NC = 2
_NS = 16
_NW = _NC * _NS
_L = 16


def _pe_tc_body(hidden, out_ref):
    rows, cols = out_ref.shape
    r = lax.broadcasted_iota(jnp.int32, (rows, cols), 0)
    c = lax.broadcasted_iota(jnp.int32, (rows, cols), 1)
    i = r * cols + c
    h = i % hidden
    pos = (i // hidden + 1).astype(jnp.float32)
    jeven = (h - (h % 2)).astype(jnp.float32)
    inv_divisor = jnp.exp(jeven * (-math.log(10000.0) / hidden))
    angle = pos * inv_divisor
    out_ref[...] = jnp.where(h % 2 == 0, jnp.sin(angle), jnp.cos(angle))


def _position_encoding(seq, hidden):
    rows = seq * hidden // 128
    return pl.pallas_call(
        functools.partial(_pe_tc_body, hidden),
        out_shape=jax.ShapeDtypeStruct((rows, 128), jnp.float32),
    )()


def _sc_body(seq, hidden, bpw, idx_hbm, table_hbm, pe_hbm, out_hbm,
             idx_v, pbufs, gbufs, obufs, gsems, osems):
    cid = lax.axis_index("c")
    sid = lax.axis_index("s")
    wid = sid * _NC + cid
    hb = hidden // 8
    nh = hidden // _L

    pltpu.sync_copy(idx_hbm.at[:, pl.ds(wid * bpw, bpw)], idx_v)

    iota = lax.broadcasted_iota(jnp.int32, (_L,), 0)
    hvecs = [hh * _L + iota for hh in range(nh)]
    h8vecs = [hv // 8 for hv in hvecs]
    hrvecs = [hv % 8 for hv in hvecs]

    def pe_slice(s):
        return pe_hbm.at[s * hidden // 128, pl.ds(s * hidden % 128, hidden)]

    def gather(s, b):
        pltpu.async_copy(table_hbm.at[idx_v.at[s]], gbufs[b], gsems[b])
        pltpu.async_copy(pe_slice(s), pbufs[b], gsems[b])

    def wait_gather(s, b):
        pltpu.make_async_copy(table_hbm.at[idx_v.at[s]], gbufs[b],
                              gsems[b]).wait()
        pltpu.make_async_copy(pe_slice(s), pbufs[b], gsems[b]).wait()

    def put(s, b):
        pltpu.async_copy(obufs[b].at[:, :, pl.ds(0, 128)],
                         out_hbm.at[s, :, wid], osems[b])

    def wait_put(s, b):
        pltpu.make_async_copy(obufs[b].at[:, :, pl.ds(0, 128)],
                              out_hbm.at[s, :, wid], osems[b]).wait()

    def transpose_pe(s, b):
        gbuf, obuf, pbuf = gbufs[b], obufs[b], pbufs[b]
        pes = [pbuf[pl.ds(hh * _L, _L)] for hh in range(nh)]

        @plsc.parallel_loop(0, bpw)
        def _(r):
            cvec = jnp.full((_L,), r, jnp.int32)
            for hh in range(nh):
                v = gbuf[r, pl.ds(hh * _L, _L)]
                plsc.store_scatter(obuf, [h8vecs[hh], hrvecs[hh], cvec],
                                   v + pes[hh])

    nbuf, lead = 4, 3
    for s0 in range(lead):
        gather(s0, s0)

    def group(g, carry):
        for b in range(nbuf):
            s = g * nbuf + b

            @pl.when(s + lead < seq)
            def _():
                gather(s + lead, (b + lead) % nbuf)

            wait_gather(s, b)

            @pl.when(s >= nbuf)
            def _():
                wait_put(s - nbuf, b)

            transpose_pe(s, b)
            put(s, b)
        return carry

    lax.fori_loop(0, seq // nbuf, group, 0, unroll=False)

    for s in range(seq - nbuf, seq):
        wait_put(s, s % nbuf)


def _sc_gather(idx_t, table, pe):
    seq, batch = idx_t.shape
    hidden = table.shape[1]
    bpw = batch // _NW
    mesh = plsc.VectorSubcoreMesh(
        core_axis_name="c", subcore_axis_name="s",
        num_cores=_NC, num_subcores=_NS)
    body = functools.partial(_sc_body, seq, hidden, bpw)
    return pl.kernel(
        body,
        out_type=jax.ShapeDtypeStruct((seq, hidden // 8, _NW, 8, 128),
                                      jnp.float32),
        mesh=mesh,
        scratch_types=[
            pltpu.VMEM((seq, bpw), jnp.int32),
            [pltpu.VMEM((hidden,), jnp.float32) for _ in range(4)],
            [pltpu.VMEM((bpw, hidden), jnp.float32) for _ in range(4)],
            [pltpu.VMEM((hidden // 8, 8, 129), jnp.float32)
             for _ in range(4)],
            [pltpu.SemaphoreType.DMA for _ in range(4)],
            [pltpu.SemaphoreType.DMA for _ in range(4)],
        ],
        compiler_params=pltpu.CompilerParams(use_tc_tiling_on_sc=False,
                                             needs_layout_passes=False),
    )(idx_t, table, pe)


def kernel(inputs, table):
    batch, seq = inputs.shape
    hidden = table.shape[1]
    pe = _position_encoding(seq, hidden)
    idx_t = inputs.T
    out5 = _sc_gather(idx_t, table, pe)
    out = out5.transpose(2, 4, 0, 1, 3).reshape(batch, seq, hidden)
    return out


# speedup vs baseline: 6.7413x; 1.0858x over previous
import functools
import math

import jax
import jax.numpy as jnp
from jax import lax
from jax.experimental import pallas as pl
from jax.experimental.pallas import tpu as pltpu
from jax.experimental.pallas import tpu_sc as plsc

_NC = 2
_NS = 16
_NW = _NC * _NS
_L = 16


def _pe_tc_body(hidden, out_ref):
    rows, cols = out_ref.shape
    r = lax.broadcasted_iota(jnp.int32, (rows, cols), 0)
    c = lax.broadcasted_iota(jnp.int32, (rows, cols), 1)
    i = r * cols + c
    h = i % hidden
    pos = (i // hidden + 1).astype(jnp.float32)
    jeven = (h - (h % 2)).astype(jnp.float32)
    inv_divisor = jnp.exp(jeven * (-math.log(10000.0) / hidden))
    angle = pos * inv_divisor
    out_ref[...] = jnp.where(h % 2 == 0, jnp.sin(angle), jnp.cos(angle))


def _position_encoding(seq, hidden):
    rows = seq * hidden // 128
    return pl.pallas_call(
        functools.partial(_pe_tc_body, hidden),
        out_shape=jax.ShapeDtypeStruct((rows, 128), jnp.float32),
    )()


def _detile_tc_body(hidden, xa_ref, xb_ref, out_ref):
    out_ref[:, 0:hidden] = xa_ref[...].T
    out_ref[:, hidden:] = xb_ref[...].T


def _detile_table(table_t, blk=1536):
    hidden, vocab = table_t.shape
    half = 128 // hidden
    grid = -(-vocab // (2 * blk))
    v2 = grid * blk
    return pl.pallas_call(
        functools.partial(_detile_tc_body, hidden),
        grid=(grid,),
        in_specs=[
            pl.BlockSpec((hidden, blk), lambda g: (0, g)),
            pl.BlockSpec((hidden, blk), lambda g, _grid=grid: (0, g + _grid)),
        ],
        out_specs=pl.BlockSpec((blk, 128), lambda g: (g, 0)),
        out_shape=jax.ShapeDtypeStruct((v2, 128), jnp.float32),
    )(table_t, table_t)


def _sc_body(seq, hidden, bpw, v2, idx_hbm, table_hbm, pe_hbm, out_hbm,
             idx_v, pbufs, gbufs, obufs, gsems, osems):
    cid = lax.axis_index("c")
    sid = lax.axis_index("s")
    wid = sid * _NC + cid
    hb = hidden // 8
    nh = hidden // _L

    pltpu.sync_copy(idx_hbm.at[:, pl.ds(wid * bpw, bpw)], idx_v)

    @plsc.parallel_loop(0, seq)
    def _(s):
        for k in range(bpw // _L):
            v = idx_v[s, pl.ds(k * _L, _L)]
            idx_v[s, pl.ds(k * _L, _L)] = (
                v + v - jnp.where(v < v2, 0, 2 * v2 - 1))

    iota = lax.broadcasted_iota(jnp.int32, (_L,), 0)
    hvecs = [hh * _L + iota for hh in range(nh)]
    h8vecs = [hv // 8 for hv in hvecs]
    hrvecs = [hv % 8 for hv in hvecs]

    def pe_slice(s):
        return pe_hbm.at[s * hidden // 128, pl.ds(s * hidden % 128, hidden)]

    def gather(s, b):
        pltpu.async_copy(table_hbm.at[idx_v.at[s]], gbufs[b], gsems[b])
        pltpu.async_copy(pe_slice(s), pbufs[b], gsems[b])

    def wait_gather(s, b):
        pltpu.make_async_copy(table_hbm.at[idx_v.at[s]], gbufs[b],
                              gsems[b]).wait()
        pltpu.make_async_copy(pe_slice(s), pbufs[b], gsems[b]).wait()

    def put(s, b):
        pltpu.async_copy(obufs[b].at[:, :, pl.ds(0, 128)],
                         out_hbm.at[s, :, wid], osems[b])

    def wait_put(s, b):
        pltpu.make_async_copy(obufs[b].at[:, :, pl.ds(0, 128)],
                              out_hbm.at[s, :, wid], osems[b]).wait()

    def transpose_pe(s, b):
        gbuf, obuf, pbuf = gbufs[b], obufs[b], pbufs[b]
        pes = [pbuf[pl.ds(hh * _L, _L)] for hh in range(nh)]

        @plsc.parallel_loop(0, bpw)
        def _(r):
            cvec = jnp.full((_L,), r, jnp.int32)
            for hh in range(nh):
                v = gbuf[r, pl.ds(hh * _L, _L)]
                plsc.store_scatter(obuf, [h8vecs[hh], hrvecs[hh], cvec],
                                   v + pes[hh])

    nbuf, lead = 4, 3
    for s0 in range(lead):
        gather(s0, s0)

    def group(g, carry):
        for b in range(nbuf):
            s = g * nbuf + b

            @pl.when(s + lead < seq)
            def _():
                gather(s + lead, (b + lead) % nbuf)

            wait_gather(s, b)

            @pl.when(s >= nbuf)
            def _():
                wait_put(s - nbuf, b)

            transpose_pe(s, b)
            put(s, b)
        return carry

    lax.fori_loop(0, seq // nbuf, group, 0, unroll=False)

    for s in range(seq - nbuf, seq):
        wait_put(s, s % nbuf)


def _sc_gather(idx_t, table, pe, v2):
    seq, batch = idx_t.shape
    hidden = table.shape[1]
    bpw = batch // _NW
    mesh = plsc.VectorSubcoreMesh(
        core_axis_name="c", subcore_axis_name="s",
        num_cores=_NC, num_subcores=_NS)
    body = functools.partial(_sc_body, seq, hidden, bpw, v2)
    return pl.kernel(
        body,
        out_type=jax.ShapeDtypeStruct((seq, hidden // 8, _NW, 8, 128),
                                      jnp.float32),
        mesh=mesh,
        scratch_types=[
            pltpu.VMEM((seq, bpw), jnp.int32),
            [pltpu.VMEM((hidden,), jnp.float32) for _ in range(4)],
            [pltpu.VMEM((bpw, hidden), jnp.float32) for _ in range(4)],
            [pltpu.VMEM((hidden // 8, 8, 129), jnp.float32)
             for _ in range(4)],
            [pltpu.SemaphoreType.DMA for _ in range(4)],
            [pltpu.SemaphoreType.DMA for _ in range(4)],
        ],
        compiler_params=pltpu.CompilerParams(use_tc_tiling_on_sc=False,
                                             needs_layout_passes=False),
    )(idx_t, table, pe)


def kernel(inputs, table):
    batch, seq = inputs.shape
    hidden = table.shape[1]
    pe = _position_encoding(seq, hidden)
    idx_t = inputs.T
    t128 = _detile_table(table.T)
    table_rm = t128.reshape(t128.shape[0] * 128 // hidden, hidden)
    out5 = _sc_gather(idx_t, table_rm, pe, t128.shape[0])
    out = out5.transpose(2, 4, 0, 1, 3).reshape(batch, seq, hidden)
    return out


# speedup vs baseline: 6.8310x; 1.0133x over previous
import functools
import math

import jax
import jax.numpy as jnp
from jax import lax
from jax.experimental import pallas as pl
from jax.experimental.pallas import tpu as pltpu
from jax.experimental.pallas import tpu_sc as plsc

_NC = 2
_NS = 16
_NW = _NC * _NS
_L = 16


def _pe_tc_body(hidden, out_ref):
    rows, cols = out_ref.shape
    r = lax.broadcasted_iota(jnp.int32, (rows, cols), 0)
    c = lax.broadcasted_iota(jnp.int32, (rows, cols), 1)
    i = r * cols + c
    h = i % hidden
    pos = (i // hidden + 1).astype(jnp.float32)
    jeven = (h - (h % 2)).astype(jnp.float32)
    inv_divisor = jnp.exp(jeven * (-math.log(10000.0) / hidden))
    angle = pos * inv_divisor
    out_ref[...] = jnp.where(h % 2 == 0, jnp.sin(angle), jnp.cos(angle))


def _position_encoding(seq, hidden):
    rows = seq * hidden // 128
    return pl.pallas_call(
        functools.partial(_pe_tc_body, hidden),
        out_shape=jax.ShapeDtypeStruct((rows, 128), jnp.float32),
    )()


def _detile_tc_body(hidden, xa_ref, xb_ref, out_ref):
    out_ref[:, 0:hidden] = xa_ref[...].T
    out_ref[:, hidden:] = xb_ref[...].T


def _detile_table(table_t, blk=1536):
    hidden, vocab = table_t.shape
    half = 128 // hidden
    grid = -(-vocab // (2 * blk))
    v2 = grid * blk
    return pl.pallas_call(
        functools.partial(_detile_tc_body, hidden),
        grid=(grid,),
        in_specs=[
            pl.BlockSpec((hidden, blk), lambda g: (0, g)),
            pl.BlockSpec((hidden, blk), lambda g, _grid=grid: (0, g + _grid)),
        ],
        out_specs=pl.BlockSpec((blk, 128), lambda g: (g, 0)),
        out_shape=jax.ShapeDtypeStruct((v2, 128), jnp.float32),
    )(table_t, table_t)


def _sc_body(seq, hidden, bpw, v2, idx_hbm, table_hbm, pe_hbm, out_hbm,
             idx_v, pbufs, gbufs, obufs, gsems, osems):
    cid = lax.axis_index("c")
    sid = lax.axis_index("s")
    wid = sid * _NC + cid
    hb = hidden // 8
    nh = hidden // _L

    pltpu.sync_copy(idx_hbm.at[:, pl.ds(wid * bpw, bpw)], idx_v)

    @plsc.parallel_loop(0, seq)
    def _(s):
        for k in range(bpw // _L):
            v = idx_v[s, pl.ds(k * _L, _L)]
            idx_v[s, pl.ds(k * _L, _L)] = (
                v + v - jnp.where(v < v2, 0, 2 * v2 - 1))

    iota = lax.broadcasted_iota(jnp.int32, (_L,), 0)
    hvecs = [hh * _L + iota for hh in range(nh)]
    h8vecs = [hv // 8 for hv in hvecs]
    hrvecs = [hv % 8 for hv in hvecs]

    def pe_slice(s):
        return pe_hbm.at[s * hidden // 128, pl.ds(s * hidden % 128, hidden)]

    half = bpw // 2

    def gather(s, b):
        pltpu.async_copy(table_hbm.at[idx_v.at[s, pl.ds(0, half)]],
                         gbufs[b].at[pl.ds(0, half), :], gsems[b])
        pltpu.async_copy(table_hbm.at[idx_v.at[s, pl.ds(half, half)]],
                         gbufs[b].at[pl.ds(half, half), :], gsems[b])
        pltpu.async_copy(pe_slice(s), pbufs[b], gsems[b])

    def wait_gather(s, b):
        pltpu.make_async_copy(table_hbm.at[idx_v.at[s, pl.ds(0, half)]],
                              gbufs[b].at[pl.ds(0, half), :],
                              gsems[b]).wait()
        pltpu.make_async_copy(table_hbm.at[idx_v.at[s, pl.ds(half, half)]],
                              gbufs[b].at[pl.ds(half, half), :],
                              gsems[b]).wait()
        pltpu.make_async_copy(pe_slice(s), pbufs[b], gsems[b]).wait()

    def put(s, b):
        pltpu.async_copy(obufs[b].at[:, :, pl.ds(0, 128)],
                         out_hbm.at[s, :, wid], osems[b])

    def wait_put(s, b):
        pltpu.make_async_copy(obufs[b].at[:, :, pl.ds(0, 128)],
                              out_hbm.at[s, :, wid], osems[b]).wait()

    def transpose_pe(s, b):
        gbuf, obuf, pbuf = gbufs[b], obufs[b], pbufs[b]
        pes = [pbuf[pl.ds(hh * _L, _L)] for hh in range(nh)]

        @plsc.parallel_loop(0, bpw)
        def _(r):
            cvec = jnp.full((_L,), r, jnp.int32)
            for hh in range(nh):
                v = gbuf[r, pl.ds(hh * _L, _L)]
                plsc.store_scatter(obuf, [h8vecs[hh], hrvecs[hh], cvec],
                                   v + pes[hh])

    nbuf, lead = 5, 4
    for s0 in range(lead):
        gather(s0, s0)

    def group(g, carry):
        for b in range(nbuf):
            s = g * nbuf + b

            @pl.when(s + lead < seq)
            def _():
                gather(s + lead, (b + lead) % nbuf)

            wait_gather(s, b)

            @pl.when(s >= nbuf)
            def _():
                wait_put(s - nbuf, b)

            transpose_pe(s, b)
            put(s, b)
        return carry

    lax.fori_loop(0, seq // nbuf, group, 0, unroll=False)

    for s in range(seq - nbuf, seq):
        wait_put(s, s % nbuf)


def _sc_gather(idx_t, table, pe, v2):
    seq, batch = idx_t.shape
    hidden = table.shape[1]
    bpw = batch // _NW
    mesh = plsc.VectorSubcoreMesh(
        core_axis_name="c", subcore_axis_name="s",
        num_cores=_NC, num_subcores=_NS)
    body = functools.partial(_sc_body, seq, hidden, bpw, v2)
    return pl.kernel(
        body,
        out_type=jax.ShapeDtypeStruct((seq, hidden // 8, _NW, 8, 128),
                                      jnp.float32),
        mesh=mesh,
        scratch_types=[
            pltpu.VMEM((seq, bpw), jnp.int32),
            [pltpu.VMEM((hidden,), jnp.float32) for _ in range(5)],
            [pltpu.VMEM((bpw, hidden), jnp.float32) for _ in range(5)],
            [pltpu.VMEM((hidden // 8, 8, 129), jnp.float32)
             for _ in range(5)],
            [pltpu.SemaphoreType.DMA for _ in range(5)],
            [pltpu.SemaphoreType.DMA for _ in range(5)],
        ],
        compiler_params=pltpu.CompilerParams(use_tc_tiling_on_sc=False,
                                             needs_layout_passes=False),
    )(idx_t, table, pe)


def kernel(inputs, table):
    batch, seq = inputs.shape
    hidden = table.shape[1]
    pe = _position_encoding(seq, hidden)
    idx_t = inputs.T
    t128 = _detile_table(table.T)
    table_rm = t128.reshape(t128.shape[0] * 128 // hidden, hidden)
    out5 = _sc_gather(idx_t, table_rm, pe, t128.shape[0])
    out = out5.transpose(2, 4, 0, 1, 3).reshape(batch, seq, hidden)
    return out
